# baseline (speedup 1.0000x reference)
import functools

import jax
import jax.numpy as jnp
from jax import lax
from jax.experimental import pallas as pl
from jax.experimental.pallas import tpu as pltpu
from jax.experimental.pallas import tpu_sc as plsc

D = 32
B = 16384
NW = 32
BPW = B // NW
CHUNK = 128
NCHUNK = BPW // CHUNK
ROWS = (1000000 * D) // 128


def _body(idx_hbm, tabr_hbm, out_hbm, idx_v, ridx_v, dst_v, rows_v, sem):
    wid = lax.axis_index("s") * 2 + lax.axis_index("c")
    base = wid * BPW
    pltpu.sync_copy(idx_hbm.at[pl.ds(base, BPW)], idx_v)

    def mkrow(v, c):
        vec = idx_v[pl.ds(v * 16, 16)]
        ridx_v[pl.ds(v * 16, 16)] = lax.shift_right_logical(vec, 2)
        return c

    lax.fori_loop(0, BPW // 16, mkrow, 0)

    copies = [
        pltpu.async_copy(
            tabr_hbm.at[ridx_v.at[pl.ds(c * CHUNK, CHUNK)]],
            dst_v.at[pl.ds(c * CHUNK, CHUNK), :],
            sem,
        )
        for c in range(NCHUNK)
    ]
    for c in copies:
        c.wait()

    iota16 = lax.iota(jnp.int32, 16)

    def extract(v, c):
        vec = idx_v[pl.ds(v * 16, 16)]
        off = (vec & 3) * D
        kv = iota16 + v * 16
        kd = kv * D
        for d in range(D):
            vals = plsc.load_gather(dst_v, [kv, off + d])
            plsc.store_scatter(rows_v, [kd + d], vals)
        return c

    lax.fori_loop(0, BPW // 16, extract, 0)
    pltpu.sync_copy(rows_v, out_hbm.at[pl.ds(base * D, BPW * D)])


_gather = functools.partial(
    pl.kernel,
    mesh=plsc.VectorSubcoreMesh(core_axis_name="c", subcore_axis_name="s"),
    out_type=jax.ShapeDtypeStruct((B * D,), jnp.float32),
    compiler_params=pltpu.CompilerParams(
        use_tc_tiling_on_sc=False, needs_layout_passes=False
    ),
    scratch_types=[
        pltpu.VMEM((BPW,), jnp.int32),
        pltpu.VMEM((BPW,), jnp.int32),
        pltpu.VMEM((BPW, 128), jnp.float32),
        pltpu.VMEM((BPW * D,), jnp.float32),
        pltpu.SemaphoreType.DMA,
    ],
)(_body)


def kernel(goal_id, table):
    tabr = table.reshape(ROWS, 128)
    flat = _gather(goal_id.astype(jnp.int32), tabr)
    return flat.reshape(B, D)


# speedup vs baseline: 1.7276x; 1.7276x over previous
import functools

import jax
import jax.numpy as jnp
from jax import lax
from jax.experimental import pallas as pl
from jax.experimental.pallas import tpu as pltpu
from jax.experimental.pallas import tpu_sc as plsc

D = 32
B = 16384
NW = 32
MAIN = 31232
WIN = 512
NWIN_MAIN = 61
NWINT = 62
EXTRA_LO = 999424
TAIL_LO = 999936
TAIL_W = 64
RBG = 8
SEL_CAP = B


def _body(idx_hbm, tab_hbm, tail_hbm, out_hbm,
          idx_v, sel_i, sel_b, wsel_i, wsel_b, win0, win1, tail_v, rowbuf,
          ssem0, ssem1, osem):
    wid = lax.axis_index("s") * 2 + lax.axis_index("c")
    lo = wid * MAIN
    hi = lo + MAIN
    iota16 = lax.iota(jnp.int32, 16)

    pltpu.sync_copy(idx_hbm, idx_v)

    def sel_body(v, cnt):
        vec = idx_v[pl.ds(v * 16, 16)]
        bvec = iota16 + v * 16
        m = ((vec >= lo) & (vec < hi)) | (vec >= EXTRA_LO)
        mi = m.astype(jnp.int32)
        pos = cnt + plsc.cumsum(mi) - 1
        plsc.store_scatter(sel_i, [pos], vec, mask=m)
        plsc.store_scatter(sel_b, [pos], bvec, mask=m)
        return cnt + jnp.sum(mi)

    cnt = lax.fori_loop(0, B // 16, sel_body, 0)
    nmatch = (cnt + 15) // 16

    wins = (win0, win1)
    ssems = (ssem0, ssem1)

    def start(w, p):
        base = pl.multiple_of(
            jnp.where(w < NWIN_MAIN, lo + w * WIN, EXTRA_LO), 128
        )
        pltpu.async_copy(tab_hbm.at[:, pl.ds(base, WIN)], wins[p], ssems[p])

    def wait_win(p):
        pltpu.make_async_copy(
            tab_hbm.at[:, pl.ds(0, WIN)], wins[p], ssems[p]
        ).wait()

    def match_extract(wbase, width, win_ref, gidx):
        def match(u, wcnt):
            iv = sel_i[pl.ds(u * 16, 16)]
            bv = sel_b[pl.ds(u * 16, 16)]
            ok = (iota16 + u * 16) < cnt
            m = ok & (iv >= wbase) & (iv < wbase + width)
            mi = m.astype(jnp.int32)
            pos = wcnt + plsc.cumsum(mi) - 1
            plsc.store_scatter(wsel_i, [pos], iv - wbase, mask=m)
            plsc.store_scatter(wsel_b, [pos], bv, mask=m)
            return wcnt + jnp.sum(mi)

        wcnt = lax.fori_loop(0, nmatch, match, 0)

        def ext(u, g):
            ivec = wsel_i[pl.ds(u * 16, 16)]
            bvec = wsel_b[pl.ds(u * 16, 16)]
            okv = (iota16 + u * 16) < wcnt
            iv2 = jnp.where(okv, ivec, 0)
            slotbase = (g % RBG) * 16 * D
            for d in range(D):
                vals = plsc.load_gather(win_ref, [iota16 * 0 + d, iv2])
                plsc.store_scatter(rowbuf, [slotbase + iota16 * D + d], vals)
            for l in range(16):
                ok_l = (u * 16 + l) < wcnt
                b = jnp.where(ok_l, bvec[l], B)
                pltpu.async_copy(
                    rowbuf.at[pl.ds(pl.multiple_of(slotbase + l * D, 8), D)],
                    out_hbm.at[pl.ds(b * D, D)],
                    osem,
                )

            @pl.when(g >= RBG - 1)
            def _():
                pltpu.make_async_copy(
                    rowbuf.at[pl.ds(0, 16 * D)],
                    out_hbm.at[pl.ds(0, 16 * D)],
                    osem,
                ).wait()

            return g + 1

        return lax.fori_loop(0, (wcnt + 15) // 16, ext, gidx)

    def do_window(w, p, gidx):
        wait_win(p)
        wbase = jnp.where(w < NWIN_MAIN, lo + w * WIN, EXTRA_LO)
        gidx = match_extract(wbase, WIN, wins[p], gidx)

        @pl.when(w + 2 < NWINT)
        def _():
            start(w + 2, p)

        return gidx

    start(0, 0)
    start(1, 1)

    def wpair(t, gidx):
        gidx = do_window(2 * t, 0, gidx)
        gidx = do_window(2 * t + 1, 1, gidx)
        return gidx

    gidx = lax.fori_loop(0, NWINT // 2, wpair, 0)

    pltpu.sync_copy(tail_hbm, tail_v)
    gidx = match_extract(TAIL_LO, TAIL_W, tail_v, gidx)

    def drain(_, c):
        pltpu.make_async_copy(
            rowbuf.at[pl.ds(0, 16 * D)],
            out_hbm.at[pl.ds(0, 16 * D)],
            osem,
        ).wait()
        return c

    lax.fori_loop(0, jnp.minimum(gidx, RBG - 1), drain, 0)


_lookup = functools.partial(
    pl.kernel,
    mesh=plsc.VectorSubcoreMesh(core_axis_name="c", subcore_axis_name="s"),
    out_type=jax.ShapeDtypeStruct((B * D + D,), jnp.float32),
    compiler_params=pltpu.CompilerParams(needs_layout_passes=False),
    scratch_types=[
        pltpu.VMEM((B,), jnp.int32),
        pltpu.VMEM((SEL_CAP,), jnp.int32),
        pltpu.VMEM((SEL_CAP,), jnp.int32),
        pltpu.VMEM((SEL_CAP,), jnp.int32),
        pltpu.VMEM((SEL_CAP,), jnp.int32),
        pltpu.VMEM((D, WIN), jnp.float32),
        pltpu.VMEM((D, WIN), jnp.float32),
        pltpu.VMEM((D, TAIL_W), jnp.float32),
        pltpu.VMEM((RBG * 16 * D,), jnp.float32),
        pltpu.SemaphoreType.DMA,
        pltpu.SemaphoreType.DMA,
        pltpu.SemaphoreType.DMA,
    ],
)(_body)


def kernel(goal_id, table):
    tab_t = table.T
    tail = lax.slice(tab_t, (0, TAIL_LO), (D, 1000000))
    flat = _lookup(goal_id.astype(jnp.int32), tab_t, tail)
    return flat[: B * D].reshape(B, D)


# speedup vs baseline: 2.5340x; 1.4668x over previous
import functools

import jax
import jax.numpy as jnp
from jax import lax
from jax.experimental import pallas as pl
from jax.experimental.pallas import tpu as pltpu
from jax.experimental.pallas import tpu_sc as plsc

D = 32
B = 16384
NW = 32
WIN = 1024
NWIN_MAIN = 30
MAIN = NWIN_MAIN * WIN
NWINT = 32
EXTRA_A0 = 983040
EXTRA_B_LO = 999424
EXTRA_B_SB = 998912
TAIL_LO = 999936
TAIL_W = 64
RBG = 8


def _body(idx_hbm, tab_hbm, tail_hbm, out_hbm,
          idx_v, sel_b, wsel_b, win0, win1, tail_v, rowbuf,
          ssem0, ssem1, osem):
    wid = lax.axis_index("s") * 2 + lax.axis_index("c")
    lo = wid * MAIN
    hi = lo + MAIN
    elo = jnp.where(wid < 16, EXTRA_A0 + wid * WIN, 0)
    ehi = jnp.where(wid < 16, elo + WIN, 0)
    iota16 = lax.iota(jnp.int32, 16)

    pltpu.sync_copy(idx_hbm, idx_v)

    def sel_body(v, cnt):
        vec = idx_v[pl.ds(v * 16, 16)]
        m = ((vec >= lo) & (vec < hi)) | ((vec >= elo) & (vec < ehi)) \
            | (vec >= EXTRA_B_LO)
        nm = plsc.all_reduce_population_count(m)[0]

        @pl.when(nm > 0)
        def _():
            pos = cnt + plsc.cumsum(m.astype(jnp.int32)) - 1
            plsc.store_scatter(sel_b, [pos], iota16 + v * 16, mask=m)

        return cnt + nm

    cnt = lax.fori_loop(0, B // 16, sel_body, 0, unroll=2)
    nmatch = (cnt + 15) // 16

    wins = (win0, win1)
    ssems = (ssem0, ssem1)

    def params_of(w):
        mlo = jnp.where(w < NWIN_MAIN, lo + w * WIN,
                        jnp.where(w == NWIN_MAIN, elo, EXTRA_B_LO))
        mhi = jnp.where(w < NWIN_MAIN, lo + w * WIN + WIN,
                        jnp.where(w == NWIN_MAIN, ehi, TAIL_LO))
        sbase = jnp.where(w < NWIN_MAIN, lo + w * WIN,
                          jnp.where(w == NWIN_MAIN, elo, EXTRA_B_SB))
        return mlo, mhi, sbase

    def start(w, p):
        _, _, sbase = params_of(w)
        pltpu.async_copy(
            tab_hbm.at[:, pl.ds(pl.multiple_of(sbase, 128), WIN)],
            wins[p], ssems[p],
        )

    def match_extract(sbase, mlo, mhi, win_ref, gidx):
        def match(u, wcnt):
            ok = (iota16 + u * 16) < cnt
            bv = jnp.where(ok, sel_b[pl.ds(u * 16, 16)], 0)
            iv = plsc.load_gather(idx_v, [bv])
            m = ok & (iv >= mlo) & (iv < mhi)
            nm = plsc.all_reduce_population_count(m)[0]

            @pl.when(nm > 0)
            def _():
                pos = wcnt + plsc.cumsum(m.astype(jnp.int32)) - 1
                plsc.store_scatter(wsel_b, [pos], bv, mask=m)

            return wcnt + nm

        wcnt = lax.fori_loop(0, nmatch, match, 0)

        def ext(u, g):
            okv = (iota16 + u * 16) < wcnt
            bvec = jnp.where(okv, wsel_b[pl.ds(u * 16, 16)], 0)
            iv2 = jnp.where(okv, plsc.load_gather(idx_v, [bvec]) - sbase, 0)
            slotbase = (g % RBG) * 16 * D
            for d in range(D):
                vals = plsc.load_gather(win_ref, [iota16 * 0 + d, iv2])
                plsc.store_scatter(rowbuf, [slotbase + iota16 * D + d], vals)
            for l in range(16):
                ok_l = (u * 16 + l) < wcnt
                b = jnp.where(ok_l, bvec[l], B)
                pltpu.async_copy(
                    rowbuf.at[pl.ds(pl.multiple_of(slotbase + l * D, 8), D)],
                    out_hbm.at[pl.ds(b * D, D)],
                    osem,
                )

            @pl.when(g >= RBG - 1)
            def _():
                pltpu.make_async_copy(
                    rowbuf.at[pl.ds(0, 16 * D)],
                    out_hbm.at[pl.ds(0, 16 * D)],
                    osem,
                ).wait()

            return g + 1

        return lax.fori_loop(0, (wcnt + 15) // 16, ext, gidx)

    def do_window(w, p, gidx):
        pltpu.make_async_copy(
            tab_hbm.at[:, pl.ds(0, WIN)], wins[p], ssems[p]
        ).wait()
        mlo, mhi, sbase = params_of(w)
        gidx = match_extract(sbase, mlo, mhi, wins[p], gidx)

        @pl.when(w + 2 < NWINT)
        def _():
            start(w + 2, p)

        return gidx

    start(0, 0)
    start(1, 1)

    def wpair(t, gidx):
        gidx = do_window(2 * t, 0, gidx)
        gidx = do_window(2 * t + 1, 1, gidx)
        return gidx

    gidx = lax.fori_loop(0, NWINT // 2, wpair, 0)

    pltpu.sync_copy(tail_hbm, tail_v)
    gidx = match_extract(TAIL_LO, TAIL_LO, 1000000, tail_v, gidx)

    def drain(_, c):
        pltpu.make_async_copy(
            rowbuf.at[pl.ds(0, 16 * D)],
            out_hbm.at[pl.ds(0, 16 * D)],
            osem,
        ).wait()
        return c

    lax.fori_loop(0, jnp.minimum(gidx, RBG - 1), drain, 0)


_lookup = functools.partial(
    pl.kernel,
    mesh=plsc.VectorSubcoreMesh(core_axis_name="c", subcore_axis_name="s"),
    out_type=jax.ShapeDtypeStruct((B * D + D,), jnp.float32),
    compiler_params=pltpu.CompilerParams(needs_layout_passes=False),
    scratch_types=[
        pltpu.VMEM((B,), jnp.int32),
        pltpu.VMEM((B,), jnp.int32),
        pltpu.VMEM((B,), jnp.int32),
        pltpu.VMEM((D, WIN), jnp.float32),
        pltpu.VMEM((D, WIN), jnp.float32),
        pltpu.VMEM((D, TAIL_W), jnp.float32),
        pltpu.VMEM((RBG * 16 * D,), jnp.float32),
        pltpu.SemaphoreType.DMA,
        pltpu.SemaphoreType.DMA,
        pltpu.SemaphoreType.DMA,
    ],
)(_body)


def kernel(goal_id, table):
    tab_t = table.T
    tail = lax.slice(tab_t, (0, TAIL_LO), (D, 1000000))
    flat = _lookup(goal_id.astype(jnp.int32), tab_t, tail)
    return flat[: B * D].reshape(B, D)
